# baseline (speedup 1.0000x reference)
import functools

import jax
import jax.numpy as jnp
from jax import lax
from jax.experimental import pallas as pl
from jax.experimental.pallas import tpu as pltpu
from jax.experimental.pallas import tpu_sc as plsc

N = 10000
E = 320000
L = 128
STEPS = 2
HALVES = 2
EH = E // HALVES

NC = 2
NS = 16
NW = NC * NS
PER_W = EH // NW
CHUNK = 40
N_CHUNKS = PER_W // CHUNK
GPER_W = E // NW
GCH = 80
GNC = GPER_W // GCH
N_PAD = 10240
ROWS_PER_SUB = N_PAD // NS

EB = 2000
NB = 2000


@functools.cache
def _sc_kernels():
    mesh = plsc.VectorSubcoreMesh(core_axis_name="c", subcore_axis_name="s",
                                  num_cores=NC, num_subcores=NS)

    @functools.partial(
        pl.kernel,
        out_type=(jax.ShapeDtypeStruct((E, L), jnp.float32),
                  jax.ShapeDtypeStruct((E, L), jnp.float32)),
        mesh=mesh,
        scratch_types=[
            pltpu.VMEM((GNC, GCH), jnp.int32),
            pltpu.VMEM((GNC, GCH), jnp.int32),
            pltpu.VMEM((4, GCH, L), jnp.float32),
            pltpu.VMEM((4, GCH, L), jnp.float32),
            pltpu.SemaphoreType.DMA((4,)),
            pltpu.SemaphoreType.DMA((4,)),
            pltpu.SemaphoreType.DMA((4,)),
            pltpu.SemaphoreType.DMA((4,)),
        ],
    )
    def _sc_gather(ps_hbm, pr_hbm, sidx_hbm, ridx_hbm, sp_hbm, rp_hbm,
                   sidx_v, ridx_v, srows_v, rrows_v,
                   gs_sem, gr_sem, ss_sem, sr_sem):
        cid = lax.axis_index("c")
        sid = lax.axis_index("s")
        wid = sid * NC + cid
        base0 = wid * GPER_W

        pltpu.sync_copy(sidx_hbm.at[wid], sidx_v)
        pltpu.sync_copy(ridx_hbm.at[wid], ridx_v)

        def start_gather(i, b):
            pltpu.async_copy(ps_hbm.at[sidx_v.at[i]], srows_v.at[b],
                             gs_sem.at[b])
            pltpu.async_copy(pr_hbm.at[ridx_v.at[i]], rrows_v.at[b],
                             gr_sem.at[b])

        def wait_gather(i, b):
            pltpu.make_async_copy(ps_hbm.at[sidx_v.at[i]], srows_v.at[b],
                                  gs_sem.at[b]).wait()
            pltpu.make_async_copy(pr_hbm.at[ridx_v.at[i]], rrows_v.at[b],
                                  gr_sem.at[b]).wait()

        def start_store(i, b):
            base = base0 + i * GCH
            pltpu.async_copy(srows_v.at[b], sp_hbm.at[pl.ds(base, GCH)],
                             ss_sem.at[b])
            pltpu.async_copy(rrows_v.at[b], rp_hbm.at[pl.ds(base, GCH)],
                             sr_sem.at[b])

        def wait_store(i, b):
            base = base0 + i * GCH
            pltpu.make_async_copy(srows_v.at[b],
                                  sp_hbm.at[pl.ds(base, GCH)],
                                  ss_sem.at[b]).wait()
            pltpu.make_async_copy(rrows_v.at[b],
                                  rp_hbm.at[pl.ds(base, GCH)],
                                  sr_sem.at[b]).wait()

        start_gather(0, 0)
        start_gather(1, 1)
        start_gather(2, 2)

        def step(i, b, carry):
            fb = (b + 3) % 4

            @pl.when(i + 3 < N_CHUNKS)
            def _():
                @pl.when(i >= 1)
                def _():
                    wait_store(i - 1, fb)
                start_gather(i + 3, fb)

            wait_gather(i, b)
            start_store(i, b)
            return carry

        def quad(k, carry):
            for b in range(4):
                carry = step(4 * k + b, b, carry)
            return carry

        n_quads = (N_CHUNKS - 5 + 3) // 4
        lax.fori_loop(0, n_quads, quad, 0)
        for i in range(n_quads * 4, N_CHUNKS):
            b = i % 4
            fb = (b + 3) % 4
            if i + 3 < N_CHUNKS:
                if i >= 1:
                    wait_store(i - 1, fb)
                start_gather(i + 3, fb)
            wait_gather(i, b)
            start_store(i, b)
        for i in range(max(n_quads * 4, N_CHUNKS - 4), N_CHUNKS):
            wait_store(i, i % 4)

    @functools.partial(
        pl.kernel,
        out_type=jax.ShapeDtypeStruct((NC, N_PAD, L), jnp.float32),
        mesh=mesh,
        scratch_types=[
            pltpu.VMEM((N_CHUNKS, CHUNK), jnp.int32),
            pltpu.VMEM((4, CHUNK, L), jnp.float32),
            pltpu.VMEM_SHARED((N_PAD, L), jnp.float32),
            pltpu.SemaphoreType.DMA((4,)),
            pltpu.SemaphoreType.DMA((4,)),
        ],
    )
    def _sc_scatter(edges_hbm, ridx_hbm, zeros_hbm, out_hbm,
                    idx_v, rows_v, agg_sh, ld_sem, sc_sem):
        cid = lax.axis_index("c")
        sid = lax.axis_index("s")
        wid = sid * NC + cid
        base0 = wid * PER_W
        my_rows = pl.ds(sid * ROWS_PER_SUB, ROWS_PER_SUB)

        pltpu.sync_copy(zeros_hbm, agg_sh.at[my_rows])
        pltpu.sync_copy(ridx_hbm.at[wid], idx_v)
        plsc.subcore_barrier()

        def start_load(i, b):
            base = base0 + i * CHUNK
            pltpu.async_copy(edges_hbm.at[pl.ds(base, CHUNK)], rows_v.at[b],
                             ld_sem.at[b])

        def wait_load(i, b):
            base = base0 + i * CHUNK
            pltpu.make_async_copy(edges_hbm.at[pl.ds(base, CHUNK)],
                                  rows_v.at[b], ld_sem.at[b]).wait()

        def start_scat(i, b):
            pltpu.async_copy(rows_v.at[b], agg_sh.at[idx_v.at[i]],
                             sc_sem.at[b], add=True)

        def wait_scat(i, b):
            pltpu.make_async_copy(rows_v.at[b], agg_sh.at[idx_v.at[i]],
                                  sc_sem.at[b]).wait()

        start_load(0, 0)
        start_load(1, 1)
        start_load(2, 2)

        def step(i, b, carry):
            fb = (b + 3) % 4

            @pl.when(i + 3 < N_CHUNKS)
            def _():
                @pl.when(i >= 1)
                def _():
                    wait_scat(i - 1, fb)
                start_load(i + 3, fb)

            wait_load(i, b)
            start_scat(i, b)
            return carry

        def quad(k, carry):
            for b in range(4):
                carry = step(4 * k + b, b, carry)
            return carry

        n_quads = (N_CHUNKS - 5 + 3) // 4
        lax.fori_loop(0, n_quads, quad, 0)
        for i in range(n_quads * 4, N_CHUNKS):
            b = i % 4
            fb = (b + 3) % 4
            if i + 3 < N_CHUNKS:
                if i >= 1:
                    wait_scat(i - 1, fb)
                start_load(i + 3, fb)
            wait_load(i, b)
            start_scat(i, b)
        for i in range(max(n_quads * 4, N_CHUNKS - 4), N_CHUNKS):
            wait_scat(i, i % 4)
        plsc.subcore_barrier()
        pltpu.sync_copy(agg_sh.at[my_rows], out_hbm.at[cid, my_rows])

    return _sc_gather, _sc_scatter


def _ln(x, g, b):
    m = jnp.mean(x, axis=-1, keepdims=True)
    xc = x - m
    v = jnp.mean(xc * xc, axis=-1, keepdims=True)
    return xc * lax.rsqrt(v + 1e-5) * g + b


def _mm(a, b):
    return jnp.dot(a, b, preferred_element_type=jnp.float32)


def _enc_body(x_ref, w1_ref, b1_ref, w2_ref, b2_ref, g_ref, bb_ref, o_ref):
    h = jnp.maximum(_mm(x_ref[...], w1_ref[...]) + b1_ref[...], 0.0)
    u = _mm(h, w2_ref[...]) + b2_ref[...]
    o_ref[...] = _ln(u, g_ref[...], bb_ref[...])


def _enc_call(x, w1, b1, w2, b2, g, bb, blk):
    n, d = x.shape
    grid = (n // blk,)
    full = lambda a: pl.BlockSpec(a.shape, lambda i: (0,) * a.ndim)
    return pl.pallas_call(
        _enc_body,
        grid=grid,
        in_specs=[pl.BlockSpec((blk, d), lambda i: (i, 0)),
                  full(w1), full(b1), full(w2), full(b2), full(g), full(bb)],
        out_specs=pl.BlockSpec((blk, L), lambda i: (i, 0)),
        out_shape=jax.ShapeDtypeStruct((n, L), jnp.float32),
    )(x, w1, b1, w2, b2, g, bb)


def _edge_body(e_ref, sp_ref, rp_ref, w1a_ref, b1_ref,
               w2_ref, b2_ref, g_ref, bb_ref, o_ref):
    h = _mm(e_ref[...], w1a_ref[...]) + (sp_ref[...] + rp_ref[...])
    h = jnp.maximum(h + b1_ref[...], 0.0)
    u = _mm(h, w2_ref[...]) + b2_ref[...]
    o_ref[...] = e_ref[...] + _ln(u, g_ref[...], bb_ref[...])


def _edge_call(e, sp, rp, h, w1a, b1, w2, b2, g, bb):
    ne = e.shape[0]
    grid = (ne // EB,)
    off = h * (EH // EB)
    blk = lambda: pl.BlockSpec((EB, L), lambda i: (i, 0))
    oblk = lambda: pl.BlockSpec((EB, L), lambda i: (i + off, 0))
    full = lambda a: pl.BlockSpec(a.shape, lambda i: (0,) * a.ndim)
    return pl.pallas_call(
        _edge_body,
        grid=grid,
        in_specs=[blk(), oblk(), oblk(),
                  full(w1a), full(b1),
                  full(w2), full(b2), full(g), full(bb)],
        out_specs=blk(),
        out_shape=jax.ShapeDtypeStruct((ne, L), jnp.float32),
    )(e, sp, rp, w1a, b1, w2, b2, g, bb)


def _encp_body(x_ref, w1_ref, b1_ref, w2_ref, b2_ref, g_ref, bb_ref,
               pw1_ref, pw2_ref, o_ref, op1_ref, op2_ref):
    h = jnp.maximum(_mm(x_ref[...], w1_ref[...]) + b1_ref[...], 0.0)
    u = _mm(h, w2_ref[...]) + b2_ref[...]
    nodes = _ln(u, g_ref[...], bb_ref[...])
    o_ref[...] = nodes
    op1_ref[...] = _mm(nodes, pw1_ref[...])
    op2_ref[...] = _mm(nodes, pw2_ref[...])


def _encp_call(x, w1, b1, w2, b2, g, bb, pw1, pw2):
    n, d = x.shape
    grid = (n // NB,)
    blk = pl.BlockSpec((NB, L), lambda i: (i, 0))
    full = lambda a: pl.BlockSpec(a.shape, lambda i: (0,) * a.ndim)
    sh = jax.ShapeDtypeStruct((n, L), jnp.float32)
    return pl.pallas_call(
        _encp_body,
        grid=grid,
        in_specs=[pl.BlockSpec((NB, d), lambda i: (i, 0)),
                  full(w1), full(b1), full(w2), full(b2), full(g), full(bb),
                  full(pw1), full(pw2)],
        out_specs=(blk, blk, blk),
        out_shape=(sh, sh, sh),
    )(x, w1, b1, w2, b2, g, bb, pw1, pw2)


def _node_body(n_ref, p0_ref, p1_ref, w1a_ref, w1b_ref, b1_ref,
               w2_ref, b2_ref, g_ref, bb_ref, pw1_ref, pw2_ref,
               o_ref, op1_ref, op2_ref):
    agg = (p0_ref[0] + p0_ref[1]) + (p1_ref[0] + p1_ref[1])
    h = _mm(n_ref[...], w1a_ref[...]) + _mm(agg, w1b_ref[...])
    h = jnp.maximum(h + b1_ref[...], 0.0)
    u = _mm(h, w2_ref[...]) + b2_ref[...]
    nodes = n_ref[...] + _ln(u, g_ref[...], bb_ref[...])
    o_ref[...] = nodes
    op1_ref[...] = _mm(nodes, pw1_ref[...])
    op2_ref[...] = _mm(nodes, pw2_ref[...])


def _node_call(nodes, p0, p1, w1a, w1b, b1, w2, b2, g, bb, pw1, pw2):
    grid = (N // NB,)
    blk = pl.BlockSpec((NB, L), lambda i: (i, 0))
    pblk = pl.BlockSpec((NC, NB, L), lambda i: (0, i, 0))
    full = lambda a: pl.BlockSpec(a.shape, lambda i: (0,) * a.ndim)
    sh = jax.ShapeDtypeStruct((N, L), jnp.float32)
    return pl.pallas_call(
        _node_body,
        grid=grid,
        in_specs=[blk, pblk, pblk,
                  full(w1a), full(w1b), full(b1), full(w2), full(b2),
                  full(g), full(bb), full(pw1), full(pw2)],
        out_specs=(blk, blk, blk),
        out_shape=(sh, sh, sh),
    )(nodes, p0, p1, w1a, w1b, b1, w2, b2, g, bb, pw1, pw2)


def _node_dec_body(n_ref, p0_ref, p1_ref, w1a_ref, w1b_ref, b1_ref,
                   w2_ref, b2_ref, g_ref, bb_ref,
                   dw1_ref, db1_ref, dw2_ref, db2_ref, o_ref):
    agg = (p0_ref[0] + p0_ref[1]) + (p1_ref[0] + p1_ref[1])
    h = _mm(n_ref[...], w1a_ref[...]) + _mm(agg, w1b_ref[...])
    h = jnp.maximum(h + b1_ref[...], 0.0)
    u = _mm(h, w2_ref[...]) + b2_ref[...]
    nodes = n_ref[...] + _ln(u, g_ref[...], bb_ref[...])
    hd = jnp.maximum(_mm(nodes, dw1_ref[...]) + db1_ref[...], 0.0)
    o_ref[...] = _mm(hd, dw2_ref[...]) + db2_ref[...]


def _node_dec_call(nodes, p0, p1, w1a, w1b, b1, w2, b2, g, bb,
                   dw1, db1, dw2, db2):
    grid = (N // NB,)
    blk = pl.BlockSpec((NB, L), lambda i: (i, 0))
    pblk = pl.BlockSpec((NC, NB, L), lambda i: (0, i, 0))
    full = lambda a: pl.BlockSpec(a.shape, lambda i: (0,) * a.ndim)
    return pl.pallas_call(
        _node_dec_body,
        grid=grid,
        in_specs=[blk, pblk, pblk,
                  full(w1a), full(w1b), full(b1), full(w2), full(b2),
                  full(g), full(bb),
                  full(dw1), full(db1), full(dw2), full(db2)],
        out_specs=blk,
        out_shape=jax.ShapeDtypeStruct((N, L), jnp.float32),
    )(nodes, p0, p1, w1a, w1b, b1, w2, b2, g, bb,
      dw1, db1, dw2, db2)


def kernel(node_features, edge_features, edge_index,
           enc_node_W1, enc_node_b1, enc_node_W2, enc_node_b2,
           enc_node_ln_g, enc_node_ln_b,
           enc_edge_W1, enc_edge_b1, enc_edge_W2, enc_edge_b2,
           enc_edge_ln_g, enc_edge_ln_b,
           proc_edge_W1, proc_edge_b1, proc_edge_W2, proc_edge_b2,
           proc_edge_ln_g, proc_edge_ln_b,
           proc_node_W1, proc_node_b1, proc_node_W2, proc_node_b2,
           proc_node_ln_g, proc_node_ln_b,
           dec_W1, dec_b1, dec_W2, dec_b2):
    sc_gather, sc_scatter = _sc_kernels()
    row = lambda v: v.reshape(1, L)
    senders = edge_index[0].astype(jnp.int32)
    receivers = edge_index[1].astype(jnp.int32)
    sidx3g = senders.reshape(NW, GNC, GCH)
    ridx3g = receivers.reshape(NW, GNC, GCH)
    ridx3 = [receivers[h * EH:(h + 1) * EH].reshape(NW, N_CHUNKS, CHUNK)
             for h in range(HALVES)]
    zeros = jnp.zeros((ROWS_PER_SUB, L), jnp.float32)

    ew1_0 = proc_edge_W1[0]
    ew1_1 = proc_edge_W1[1]
    nodes, ps, pr = _encp_call(node_features, enc_node_W1, row(enc_node_b1),
                               enc_node_W2, row(enc_node_b2),
                               row(enc_node_ln_g), row(enc_node_ln_b),
                               ew1_0[L:2 * L], ew1_0[2 * L:3 * L])
    edges = [_enc_call(edge_features[h * EH:(h + 1) * EH],
                       enc_edge_W1, row(enc_edge_b1),
                       enc_edge_W2, row(enc_edge_b2),
                       row(enc_edge_ln_g), row(enc_edge_ln_b), EB)
             for h in range(HALVES)]

    for i in range(STEPS):
        ew1 = proc_edge_W1[i]
        nw1 = proc_node_W1[i]
        partials = []
        sp, rp = sc_gather(ps, pr, sidx3g, ridx3g)
        for h in range(HALVES):
            edges[h] = _edge_call(edges[h], sp, rp, h, ew1[0:L],
                                  row(proc_edge_b1[i]), proc_edge_W2[i],
                                  row(proc_edge_b2[i]),
                                  row(proc_edge_ln_g[i]),
                                  row(proc_edge_ln_b[i]))
            partials.append(sc_scatter(edges[h], ridx3[h], zeros))
        if i < STEPS - 1:
            nodes, ps, pr = _node_call(nodes, partials[0], partials[1],
                                       nw1[0:L], nw1[L:2 * L],
                                       row(proc_node_b1[i]), proc_node_W2[i],
                                       row(proc_node_b2[i]),
                                       row(proc_node_ln_g[i]),
                                       row(proc_node_ln_b[i]),
                                       ew1_1[L:2 * L], ew1_1[2 * L:3 * L])
        else:
            out = _node_dec_call(nodes, partials[0], partials[1],
                                 nw1[0:L], nw1[L:2 * L],
                                 row(proc_node_b1[i]), proc_node_W2[i],
                                 row(proc_node_b2[i]),
                                 row(proc_node_ln_g[i]),
                                 row(proc_node_ln_b[i]),
                                 dec_W1, row(dec_b1), dec_W2, row(dec_b2))
    return out


# speedup vs baseline: 1.0296x; 1.0296x over previous
import functools

import jax
import jax.numpy as jnp
from jax import lax
from jax.experimental import pallas as pl
from jax.experimental.pallas import tpu as pltpu
from jax.experimental.pallas import tpu_sc as plsc

N = 10000
E = 320000
L = 128
STEPS = 2
HALVES = 2
EH = E // HALVES

NC = 2
NS = 16
NW = NC * NS
PER_W = EH // NW
CHUNK = 40
N_CHUNKS = PER_W // CHUNK
N_PAD = 10240
ROWS_PER_SUB = N_PAD // NS

EB = 2000
NB = 2000


@functools.cache
def _sc_kernels():
    mesh = plsc.VectorSubcoreMesh(core_axis_name="c", subcore_axis_name="s",
                                  num_cores=NC, num_subcores=NS)

    @functools.partial(
        pl.kernel,
        out_type=jax.ShapeDtypeStruct((EH, L), jnp.float32),
        mesh=mesh,
        scratch_types=[
            pltpu.VMEM((N_CHUNKS, CHUNK), jnp.int32),
            pltpu.VMEM((N_CHUNKS, CHUNK), jnp.int32),
            pltpu.VMEM((4, CHUNK, L), jnp.float32),
            pltpu.VMEM((4, CHUNK, L), jnp.float32),
            pltpu.SemaphoreType.DMA((4,)),
            pltpu.SemaphoreType.DMA((4,)),
            pltpu.SemaphoreType.DMA((4,)),
        ],
    )
    def _sc_gather(ps_hbm, pr_hbm, sidx_hbm, ridx_hbm, hpre_hbm,
                   sidx_v, ridx_v, srows_v, rrows_v,
                   gs_sem, gr_sem, st_sem):
        cid = lax.axis_index("c")
        sid = lax.axis_index("s")
        wid = sid * NC + cid
        base0 = wid * PER_W

        pltpu.sync_copy(sidx_hbm.at[wid], sidx_v)
        pltpu.sync_copy(ridx_hbm.at[wid], ridx_v)

        def start_gather(i, b):
            pltpu.async_copy(ps_hbm.at[sidx_v.at[i]], srows_v.at[b],
                             gs_sem.at[b])
            pltpu.async_copy(pr_hbm.at[ridx_v.at[i]], rrows_v.at[b],
                             gr_sem.at[b])

        def wait_gather(i, b):
            pltpu.make_async_copy(ps_hbm.at[sidx_v.at[i]], srows_v.at[b],
                                  gs_sem.at[b]).wait()
            pltpu.make_async_copy(pr_hbm.at[ridx_v.at[i]], rrows_v.at[b],
                                  gr_sem.at[b]).wait()

        def add_rows(b):
            def abody(r, carry):
                for cc in range(0, L, 16):
                    srows_v[b, r, pl.ds(cc, 16)] = (
                        srows_v[b, r, pl.ds(cc, 16)]
                        + rrows_v[b, r, pl.ds(cc, 16)])
                return carry
            lax.fori_loop(0, CHUNK, abody, 0, unroll=4)

        def start_store(i, b):
            base = base0 + i * CHUNK
            pltpu.async_copy(srows_v.at[b], hpre_hbm.at[pl.ds(base, CHUNK)],
                             st_sem.at[b])

        def wait_store(i, b):
            base = base0 + i * CHUNK
            pltpu.make_async_copy(srows_v.at[b],
                                  hpre_hbm.at[pl.ds(base, CHUNK)],
                                  st_sem.at[b]).wait()

        start_gather(0, 0)
        start_gather(1, 1)
        start_gather(2, 2)

        def step(i, b, carry):
            fb = (b + 3) % 4

            @pl.when(i + 3 < N_CHUNKS)
            def _():
                @pl.when(i >= 1)
                def _():
                    wait_store(i - 1, fb)
                start_gather(i + 3, fb)

            wait_gather(i, b)
            add_rows(b)
            start_store(i, b)
            return carry

        def quad(k, carry):
            for b in range(4):
                carry = step(4 * k + b, b, carry)
            return carry

        n_quads = (N_CHUNKS - 5 + 3) // 4
        lax.fori_loop(0, n_quads, quad, 0)
        for i in range(n_quads * 4, N_CHUNKS):
            b = i % 4
            fb = (b + 3) % 4
            if i + 3 < N_CHUNKS:
                if i >= 1:
                    wait_store(i - 1, fb)
                start_gather(i + 3, fb)
            wait_gather(i, b)
            add_rows(b)
            start_store(i, b)
        for i in range(max(n_quads * 4, N_CHUNKS - 4), N_CHUNKS):
            wait_store(i, i % 4)

    @functools.partial(
        pl.kernel,
        out_type=jax.ShapeDtypeStruct((NC, N_PAD, L), jnp.float32),
        mesh=mesh,
        scratch_types=[
            pltpu.VMEM((N_CHUNKS, CHUNK), jnp.int32),
            pltpu.VMEM((4, CHUNK, L), jnp.float32),
            pltpu.VMEM_SHARED((N_PAD, L), jnp.float32),
            pltpu.SemaphoreType.DMA((4,)),
            pltpu.SemaphoreType.DMA((4,)),
        ],
    )
    def _sc_scatter(edges_hbm, ridx_hbm, zeros_hbm, out_hbm,
                    idx_v, rows_v, agg_sh, ld_sem, sc_sem):
        cid = lax.axis_index("c")
        sid = lax.axis_index("s")
        wid = sid * NC + cid
        base0 = wid * PER_W
        my_rows = pl.ds(sid * ROWS_PER_SUB, ROWS_PER_SUB)

        pltpu.sync_copy(zeros_hbm, agg_sh.at[my_rows])
        pltpu.sync_copy(ridx_hbm.at[wid], idx_v)
        plsc.subcore_barrier()

        def start_load(i, b):
            base = base0 + i * CHUNK
            pltpu.async_copy(edges_hbm.at[pl.ds(base, CHUNK)], rows_v.at[b],
                             ld_sem.at[b])

        def wait_load(i, b):
            base = base0 + i * CHUNK
            pltpu.make_async_copy(edges_hbm.at[pl.ds(base, CHUNK)],
                                  rows_v.at[b], ld_sem.at[b]).wait()

        def start_scat(i, b):
            pltpu.async_copy(rows_v.at[b], agg_sh.at[idx_v.at[i]],
                             sc_sem.at[b], add=True)

        def wait_scat(i, b):
            pltpu.make_async_copy(rows_v.at[b], agg_sh.at[idx_v.at[i]],
                                  sc_sem.at[b]).wait()

        start_load(0, 0)
        start_load(1, 1)
        start_load(2, 2)

        def step(i, b, carry):
            fb = (b + 3) % 4

            @pl.when(i + 3 < N_CHUNKS)
            def _():
                @pl.when(i >= 1)
                def _():
                    wait_scat(i - 1, fb)
                start_load(i + 3, fb)

            wait_load(i, b)
            start_scat(i, b)
            return carry

        def quad(k, carry):
            for b in range(4):
                carry = step(4 * k + b, b, carry)
            return carry

        n_quads = (N_CHUNKS - 5 + 3) // 4
        lax.fori_loop(0, n_quads, quad, 0)
        for i in range(n_quads * 4, N_CHUNKS):
            b = i % 4
            fb = (b + 3) % 4
            if i + 3 < N_CHUNKS:
                if i >= 1:
                    wait_scat(i - 1, fb)
                start_load(i + 3, fb)
            wait_load(i, b)
            start_scat(i, b)
        for i in range(max(n_quads * 4, N_CHUNKS - 4), N_CHUNKS):
            wait_scat(i, i % 4)
        plsc.subcore_barrier()
        pltpu.sync_copy(agg_sh.at[my_rows], out_hbm.at[cid, my_rows])

    return _sc_gather, _sc_scatter


def _ln(x, g, b):
    m = jnp.mean(x, axis=-1, keepdims=True)
    xc = x - m
    v = jnp.mean(xc * xc, axis=-1, keepdims=True)
    return xc * lax.rsqrt(v + 1e-5) * g + b


def _mm(a, b):
    return jnp.dot(a, b, preferred_element_type=jnp.float32)


def _enc_body(x_ref, w1_ref, b1_ref, w2_ref, b2_ref, g_ref, bb_ref, o_ref):
    h = jnp.maximum(_mm(x_ref[...], w1_ref[...]) + b1_ref[...], 0.0)
    u = _mm(h, w2_ref[...]) + b2_ref[...]
    o_ref[...] = _ln(u, g_ref[...], bb_ref[...])


def _enc_call(x, w1, b1, w2, b2, g, bb, blk):
    n, d = x.shape
    grid = (n // blk,)
    full = lambda a: pl.BlockSpec(a.shape, lambda i: (0,) * a.ndim)
    return pl.pallas_call(
        _enc_body,
        grid=grid,
        in_specs=[pl.BlockSpec((blk, d), lambda i: (i, 0)),
                  full(w1), full(b1), full(w2), full(b2), full(g), full(bb)],
        out_specs=pl.BlockSpec((blk, L), lambda i: (i, 0)),
        out_shape=jax.ShapeDtypeStruct((n, L), jnp.float32),
    )(x, w1, b1, w2, b2, g, bb)


def _edge_body(e_ref, hp_ref, w1a_ref, b1_ref,
               w2_ref, b2_ref, g_ref, bb_ref, o_ref):
    h = _mm(e_ref[...], w1a_ref[...]) + hp_ref[...]
    h = jnp.maximum(h + b1_ref[...], 0.0)
    u = _mm(h, w2_ref[...]) + b2_ref[...]
    o_ref[...] = e_ref[...] + _ln(u, g_ref[...], bb_ref[...])


def _edge_call(e, hp, w1a, b1, w2, b2, g, bb):
    ne = e.shape[0]
    grid = (ne // EB,)
    blk = lambda: pl.BlockSpec((EB, L), lambda i: (i, 0))
    full = lambda a: pl.BlockSpec(a.shape, lambda i: (0,) * a.ndim)
    return pl.pallas_call(
        _edge_body,
        grid=grid,
        in_specs=[blk(), blk(),
                  full(w1a), full(b1),
                  full(w2), full(b2), full(g), full(bb)],
        out_specs=blk(),
        out_shape=jax.ShapeDtypeStruct((ne, L), jnp.float32),
    )(e, hp, w1a, b1, w2, b2, g, bb)


def _encp_body(x_ref, w1_ref, b1_ref, w2_ref, b2_ref, g_ref, bb_ref,
               pw1_ref, pw2_ref, o_ref, op1_ref, op2_ref):
    h = jnp.maximum(_mm(x_ref[...], w1_ref[...]) + b1_ref[...], 0.0)
    u = _mm(h, w2_ref[...]) + b2_ref[...]
    nodes = _ln(u, g_ref[...], bb_ref[...])
    o_ref[...] = nodes
    op1_ref[...] = _mm(nodes, pw1_ref[...])
    op2_ref[...] = _mm(nodes, pw2_ref[...])


def _encp_call(x, w1, b1, w2, b2, g, bb, pw1, pw2):
    n, d = x.shape
    grid = (n // NB,)
    blk = pl.BlockSpec((NB, L), lambda i: (i, 0))
    full = lambda a: pl.BlockSpec(a.shape, lambda i: (0,) * a.ndim)
    sh = jax.ShapeDtypeStruct((n, L), jnp.float32)
    return pl.pallas_call(
        _encp_body,
        grid=grid,
        in_specs=[pl.BlockSpec((NB, d), lambda i: (i, 0)),
                  full(w1), full(b1), full(w2), full(b2), full(g), full(bb),
                  full(pw1), full(pw2)],
        out_specs=(blk, blk, blk),
        out_shape=(sh, sh, sh),
    )(x, w1, b1, w2, b2, g, bb, pw1, pw2)


def _node_body(n_ref, p0_ref, p1_ref, w1a_ref, w1b_ref, b1_ref,
               w2_ref, b2_ref, g_ref, bb_ref, pw1_ref, pw2_ref,
               o_ref, op1_ref, op2_ref):
    agg = (p0_ref[0] + p0_ref[1]) + (p1_ref[0] + p1_ref[1])
    h = _mm(n_ref[...], w1a_ref[...]) + _mm(agg, w1b_ref[...])
    h = jnp.maximum(h + b1_ref[...], 0.0)
    u = _mm(h, w2_ref[...]) + b2_ref[...]
    nodes = n_ref[...] + _ln(u, g_ref[...], bb_ref[...])
    o_ref[...] = nodes
    op1_ref[...] = _mm(nodes, pw1_ref[...])
    op2_ref[...] = _mm(nodes, pw2_ref[...])


def _node_call(nodes, p0, p1, w1a, w1b, b1, w2, b2, g, bb, pw1, pw2):
    grid = (N // NB,)
    blk = pl.BlockSpec((NB, L), lambda i: (i, 0))
    pblk = pl.BlockSpec((NC, NB, L), lambda i: (0, i, 0))
    full = lambda a: pl.BlockSpec(a.shape, lambda i: (0,) * a.ndim)
    sh = jax.ShapeDtypeStruct((N, L), jnp.float32)
    return pl.pallas_call(
        _node_body,
        grid=grid,
        in_specs=[blk, pblk, pblk,
                  full(w1a), full(w1b), full(b1), full(w2), full(b2),
                  full(g), full(bb), full(pw1), full(pw2)],
        out_specs=(blk, blk, blk),
        out_shape=(sh, sh, sh),
    )(nodes, p0, p1, w1a, w1b, b1, w2, b2, g, bb, pw1, pw2)


def _node_dec_body(n_ref, p0_ref, p1_ref, w1a_ref, w1b_ref, b1_ref,
                   w2_ref, b2_ref, g_ref, bb_ref,
                   dw1_ref, db1_ref, dw2_ref, db2_ref, o_ref):
    agg = (p0_ref[0] + p0_ref[1]) + (p1_ref[0] + p1_ref[1])
    h = _mm(n_ref[...], w1a_ref[...]) + _mm(agg, w1b_ref[...])
    h = jnp.maximum(h + b1_ref[...], 0.0)
    u = _mm(h, w2_ref[...]) + b2_ref[...]
    nodes = n_ref[...] + _ln(u, g_ref[...], bb_ref[...])
    hd = jnp.maximum(_mm(nodes, dw1_ref[...]) + db1_ref[...], 0.0)
    o_ref[...] = _mm(hd, dw2_ref[...]) + db2_ref[...]


def _node_dec_call(nodes, p0, p1, w1a, w1b, b1, w2, b2, g, bb,
                   dw1, db1, dw2, db2):
    grid = (N // NB,)
    blk = pl.BlockSpec((NB, L), lambda i: (i, 0))
    pblk = pl.BlockSpec((NC, NB, L), lambda i: (0, i, 0))
    full = lambda a: pl.BlockSpec(a.shape, lambda i: (0,) * a.ndim)
    return pl.pallas_call(
        _node_dec_body,
        grid=grid,
        in_specs=[blk, pblk, pblk,
                  full(w1a), full(w1b), full(b1), full(w2), full(b2),
                  full(g), full(bb),
                  full(dw1), full(db1), full(dw2), full(db2)],
        out_specs=blk,
        out_shape=jax.ShapeDtypeStruct((N, L), jnp.float32),
    )(nodes, p0, p1, w1a, w1b, b1, w2, b2, g, bb,
      dw1, db1, dw2, db2)


def kernel(node_features, edge_features, edge_index,
           enc_node_W1, enc_node_b1, enc_node_W2, enc_node_b2,
           enc_node_ln_g, enc_node_ln_b,
           enc_edge_W1, enc_edge_b1, enc_edge_W2, enc_edge_b2,
           enc_edge_ln_g, enc_edge_ln_b,
           proc_edge_W1, proc_edge_b1, proc_edge_W2, proc_edge_b2,
           proc_edge_ln_g, proc_edge_ln_b,
           proc_node_W1, proc_node_b1, proc_node_W2, proc_node_b2,
           proc_node_ln_g, proc_node_ln_b,
           dec_W1, dec_b1, dec_W2, dec_b2):
    sc_gather, sc_scatter = _sc_kernels()
    row = lambda v: v.reshape(1, L)
    senders = edge_index[0].astype(jnp.int32)
    receivers = edge_index[1].astype(jnp.int32)
    sidx3 = [senders[h * EH:(h + 1) * EH].reshape(NW, N_CHUNKS, CHUNK)
             for h in range(HALVES)]
    ridx3 = [receivers[h * EH:(h + 1) * EH].reshape(NW, N_CHUNKS, CHUNK)
             for h in range(HALVES)]
    zeros = jnp.zeros((ROWS_PER_SUB, L), jnp.float32)

    ew1_0 = proc_edge_W1[0]
    ew1_1 = proc_edge_W1[1]
    nodes, ps, pr = _encp_call(node_features, enc_node_W1, row(enc_node_b1),
                               enc_node_W2, row(enc_node_b2),
                               row(enc_node_ln_g), row(enc_node_ln_b),
                               ew1_0[L:2 * L], ew1_0[2 * L:3 * L])
    edges = [_enc_call(edge_features[h * EH:(h + 1) * EH],
                       enc_edge_W1, row(enc_edge_b1),
                       enc_edge_W2, row(enc_edge_b2),
                       row(enc_edge_ln_g), row(enc_edge_ln_b), EB)
             for h in range(HALVES)]

    for i in range(STEPS):
        ew1 = proc_edge_W1[i]
        nw1 = proc_node_W1[i]
        partials = []
        for h in range(HALVES):
            hpre = sc_gather(ps, pr, sidx3[h], ridx3[h])
            edges[h] = _edge_call(edges[h], hpre, ew1[0:L],
                                  row(proc_edge_b1[i]), proc_edge_W2[i],
                                  row(proc_edge_b2[i]),
                                  row(proc_edge_ln_g[i]),
                                  row(proc_edge_ln_b[i]))
            partials.append(sc_scatter(edges[h], ridx3[h], zeros))
        if i < STEPS - 1:
            nodes, ps, pr = _node_call(nodes, partials[0], partials[1],
                                       nw1[0:L], nw1[L:2 * L],
                                       row(proc_node_b1[i]), proc_node_W2[i],
                                       row(proc_node_b2[i]),
                                       row(proc_node_ln_g[i]),
                                       row(proc_node_ln_b[i]),
                                       ew1_1[L:2 * L], ew1_1[2 * L:3 * L])
        else:
            out = _node_dec_call(nodes, partials[0], partials[1],
                                 nw1[0:L], nw1[L:2 * L],
                                 row(proc_node_b1[i]), proc_node_W2[i],
                                 row(proc_node_b2[i]),
                                 row(proc_node_ln_g[i]),
                                 row(proc_node_ln_b[i]),
                                 dec_W1, row(dec_b1), dec_W2, row(dec_b2))
    return out
